# baseline (speedup 1.0000x reference)
import jax
import jax.numpy as jnp
from jax import lax
from jax.experimental import pallas as pl
from jax.experimental.pallas import tpu as pltpu
from jax.experimental.pallas import tpu_sc as plsc

N = 10000
E = 320000
D = 128
H = 64
C = 40
NP = 10240
NC, NS, L = 2, 16, 16
NW = NC * NS
K = 128
NSTEP = 80
EPWP = NSTEP * K
EP = NW * EPWP
RPT = NP // NS
GC = D + 16

_f32 = jnp.float32
_i32 = jnp.int32
_mesh = plsc.VectorSubcoreMesh(core_axis_name="c", subcore_axis_name="s")
_CP = pltpu.CompilerParams(needs_layout_passes=False, use_tc_tiling_on_sc=False)


def _zero_rows(rows, cols):
    for r in range(K):
        for c in range(cols // L):
            rows[r, pl.ds(c * L, L)] = jnp.zeros((L,), _f32)


def _fill_acc(rows, acc, sid):
    def cp(i, _):
        pltpu.sync_copy(rows, acc.at[pl.ds(sid * RPT + i * K, K)])
        return 0

    lax.fori_loop(0, RPT // K, cp, 0)


def _writeout(acc, rows, out, cid, sid):
    def wo(i, _):
        off = sid * RPT + i * K
        pltpu.sync_copy(acc.at[pl.ds(off, K)], rows)
        pltpu.sync_copy(rows, out.at[cid].at[pl.ds(off, K)])
        return 0

    lax.fori_loop(0, RPT // K, wo, 0)


def _sc_edge_loop(tab, edges3, acc, e0, e1, b0, b1, gs0, gs1, ss0, ss1,
                  base, scale0=None, scale1=None):
    pltpu.sync_copy(edges3.at[base], e0)
    pltpu.async_copy(tab.at[e0.at[0]], b0, gs0)

    def body(jj, _):
        c1 = base + 2 * jj + 1
        pltpu.sync_copy(edges3.at[c1], e1)
        pltpu.async_copy(tab.at[e1.at[0]], b1, gs1)
        pltpu.make_async_copy(tab.at[e0.at[0]], b0, gs0).wait()
        if scale0 is not None:
            scale0(jj, b0)
        s0 = pltpu.async_copy(b0, acc.at[e0.at[1]], ss0, add=True)
        pltpu.make_async_copy(tab.at[e1.at[0]], b1, gs1).wait()
        if scale1 is not None:
            scale1(jj, b1)
        s1 = pltpu.async_copy(b1, acc.at[e1.at[1]], ss1, add=True)
        s0.wait()
        cn = base + jnp.minimum(2 * jj + 2, NSTEP - 1)
        pltpu.sync_copy(edges3.at[cn], e0)
        pltpu.async_copy(tab.at[e0.at[0]], b0, gs0)
        s1.wait()
        return 0

    lax.fori_loop(0, NSTEP // 2, body, 0)
    pltpu.make_async_copy(tab.at[e0.at[0]], b0, gs0).wait()


def _sc_edge_loop_pre(tab, ebig, acc, b0, b1, gs0, gs1, ss0, ss1, nstep):
    pltpu.async_copy(tab.at[ebig.at[0, 0]], b0, gs0)

    def body(jj, _):
        c1 = 2 * jj + 1
        pltpu.async_copy(tab.at[ebig.at[c1, 0]], b1, gs1)
        pltpu.make_async_copy(tab.at[ebig.at[0, 0]], b0, gs0).wait()
        s0 = pltpu.async_copy(b0, acc.at[ebig.at[2 * jj, 1]], ss0, add=True)
        pltpu.make_async_copy(tab.at[ebig.at[0, 0]], b1, gs1).wait()
        s1 = pltpu.async_copy(b1, acc.at[ebig.at[c1, 1]], ss1, add=True)
        s0.wait()
        cn = jnp.minimum(2 * jj + 2, nstep - 1)
        pltpu.async_copy(tab.at[ebig.at[cn, 0]], b0, gs0)
        s1.wait()
        return 0

    lax.fori_loop(0, nstep // 2, body, 0)
    pltpu.make_async_copy(tab.at[ebig.at[0, 0]], b0, gs0).wait()


def _make_sc_spmm(cols, preload_idx=False):
    if preload_idx:
        nphase = 1 if cols <= H else 2
        pstep = NSTEP // nphase

        def body(tab, edges3, out, ebig, b0, b1, acc, gs0, gs1, ss0, ss1):
            cid = lax.axis_index("c")
            sid = lax.axis_index("s")
            base = (cid * NS + sid) * NSTEP
            _zero_rows(b0, cols)
            _fill_acc(b0, acc, sid)
            plsc.subcore_barrier()
            for ph in range(nphase):
                pltpu.sync_copy(
                    edges3.at[pl.ds(base + ph * pstep, pstep)], ebig)
                _sc_edge_loop_pre(tab, ebig, acc, b0, b1,
                                  gs0, gs1, ss0, ss1, pstep)
            plsc.subcore_barrier()
            _writeout(acc, b0, out, cid, sid)

        idx_scratch = [pltpu.VMEM((NSTEP // nphase, 2, K), _i32)]
    else:
        def body(tab, edges3, out, e0, e1, b0, b1, acc, gs0, gs1, ss0, ss1):
            cid = lax.axis_index("c")
            sid = lax.axis_index("s")
            _zero_rows(b0, cols)
            _fill_acc(b0, acc, sid)
            plsc.subcore_barrier()
            base = (cid * NS + sid) * NSTEP
            _sc_edge_loop(tab, edges3, acc, e0, e1, b0, b1,
                          gs0, gs1, ss0, ss1, base)
            plsc.subcore_barrier()
            _writeout(acc, b0, out, cid, sid)

        idx_scratch = [pltpu.VMEM((2, K), _i32), pltpu.VMEM((2, K), _i32)]

    return pl.kernel(
        body,
        out_type=jax.ShapeDtypeStruct((NC, NP, cols), _f32),
        mesh=_mesh,
        compiler_params=_CP,
        scratch_types=idx_scratch + [
            pltpu.VMEM((K, cols), _f32),
            pltpu.VMEM((K, cols), _f32),
            pltpu.VMEM_SHARED((NP, cols), _f32),
            pltpu.SemaphoreType.DMA,
            pltpu.SemaphoreType.DMA,
            pltpu.SemaphoreType.DMA,
            pltpu.SemaphoreType.DMA,
        ],
    )


def _sc_gat_body(asm, ad, mv, edges3, xaug, out,
                 e0, e1, av0, av1, dv0, dv1, b0, b1, mvv, acc,
                 gs0, gs1, as0, as1, ds0, ds1, ss0, ss1):
    cid = lax.axis_index("c")
    sid = lax.axis_index("s")
    _zero_rows(b0, GC)
    _fill_acc(b0, acc, sid)
    pltpu.sync_copy(mv, mvv)
    plsc.subcore_barrier()
    base = (cid * NS + sid) * NSTEP
    m = mvv[...]

    def issue(c, e, b, av, dv, gs, asem, dsem):
        pltpu.sync_copy(edges3.at[c], e)
        pltpu.async_copy(xaug.at[e.at[0]], b, gs)
        pltpu.async_copy(asm.at[e.at[0]], av, asem)
        pltpu.async_copy(ad.at[e.at[1]], dv, dsem)

    def tcomp(e, b, av, dv, gs, asem, dsem):
        pltpu.make_async_copy(asm.at[e.at[0]], av, asem).wait()
        pltpu.make_async_copy(ad.at[e.at[1]], dv, dsem).wait()
        for i in range(K // L):
            v = av[pl.ds(i * L, L)] + dv[pl.ds(i * L, L)]
            av[pl.ds(i * L, L)] = jnp.exp(jnp.maximum(v, 0.2 * v) - m)
        pltpu.make_async_copy(xaug.at[e.at[0]], b, gs).wait()
        for r in range(K):
            t = plsc.load_gather(av, [jnp.full((L,), r, _i32)])
            for c in range(GC // L):
                b[r, pl.ds(c * L, L)] = b[r, pl.ds(c * L, L)] * t

    issue(base, e0, b0, av0, dv0, gs0, as0, ds0)

    def body(jj, _):
        issue(base + 2 * jj + 1, e1, b1, av1, dv1, gs1, as1, ds1)
        tcomp(e0, b0, av0, dv0, gs0, as0, ds0)
        s0 = pltpu.async_copy(b0, acc.at[e0.at[1]], ss0, add=True)
        tcomp(e1, b1, av1, dv1, gs1, as1, ds1)
        s1 = pltpu.async_copy(b1, acc.at[e1.at[1]], ss1, add=True)
        s0.wait()
        cn = base + jnp.minimum(2 * jj + 2, NSTEP - 1)
        issue(cn, e0, b0, av0, dv0, gs0, as0, ds0)
        s1.wait()
        return 0

    lax.fori_loop(0, NSTEP // 2, body, 0)
    pltpu.make_async_copy(xaug.at[e0.at[0]], b0, gs0).wait()
    pltpu.make_async_copy(asm.at[e0.at[0]], av0, as0).wait()
    pltpu.make_async_copy(ad.at[e0.at[1]], dv0, ds0).wait()
    plsc.subcore_barrier()
    _writeout(acc, b0, out, cid, sid)


_sc_gat = pl.kernel(
    _sc_gat_body,
    out_type=jax.ShapeDtypeStruct((NC, NP, GC), _f32),
    mesh=_mesh,
    compiler_params=_CP,
    scratch_types=[
        pltpu.VMEM((2, K), _i32),
        pltpu.VMEM((2, K), _i32),
        pltpu.VMEM((K,), _f32),
        pltpu.VMEM((K,), _f32),
        pltpu.VMEM((K,), _f32),
        pltpu.VMEM((K,), _f32),
        pltpu.VMEM((K, GC), _f32),
        pltpu.VMEM((K, GC), _f32),
        pltpu.VMEM((L,), _f32),
        pltpu.VMEM_SHARED((NP, GC), _f32),
        pltpu.SemaphoreType.DMA,
        pltpu.SemaphoreType.DMA,
        pltpu.SemaphoreType.DMA,
        pltpu.SemaphoreType.DMA,
        pltpu.SemaphoreType.DMA,
        pltpu.SemaphoreType.DMA,
        pltpu.SemaphoreType.DMA,
        pltpu.SemaphoreType.DMA,
    ],
)


_BLK = 2048
_GRID = NP // _BLK


def _rows_spec(cols):
    return pl.BlockSpec((_BLK, cols), lambda i: (i, 0))


def _full_spec(r, c):
    return pl.BlockSpec((r, c), lambda i: (0, 0))


def _tc1a_body(x_ref, kf_ref, w1_ref, xl1_ref):
    xl1_ref[...] = jnp.dot(kf_ref[...] * x_ref[...], w1_ref[...],
                           preferred_element_type=_f32)


_tc1a = pl.pallas_call(
    _tc1a_body,
    grid=(_GRID,),
    in_specs=[_rows_spec(D), _rows_spec(1), _full_spec(D, 2 * H)],
    out_specs=_rows_spec(2 * H),
    out_shape=jax.ShapeDtypeStruct((NP, 2 * H), _f32),
)


def _tc1b_body(xl1_ref, kf_ref, d0_ref, d1_ref, u1_ref, dinv_ref):
    kf = kf_ref[...]
    deg = d0_ref[...][:, 0:1] + d1_ref[...][:, 0:1] + 1.0
    dinv = lax.rsqrt(deg)
    u1_ref[...] = (kf * dinv) * xl1_ref[...]
    dinv_ref[...] = dinv


_tc1b = pl.pallas_call(
    _tc1b_body,
    grid=(_GRID,),
    in_specs=[_rows_spec(2 * H), _rows_spec(1), _rows_spec(16),
              _rows_spec(16)],
    out_specs=[_rows_spec(2 * H), _rows_spec(1)],
    out_shape=[jax.ShapeDtypeStruct((NP, 2 * H), _f32),
               jax.ShapeDtypeStruct((NP, 1), _f32)],
)


def _tc2_body(a0_ref, a1_ref, xl1_ref, dinv_ref, kf_ref, b1_ref, w2_ref,
              xl2_ref, u2_ref):
    dinv = dinv_ref[...]
    kf = kf_ref[...]
    h1 = jax.nn.relu(dinv * (a0_ref[...] + a1_ref[...])
                     + (dinv * dinv) * xl1_ref[...] + b1_ref[...])
    xl2 = jnp.dot(h1, w2_ref[...], preferred_element_type=_f32)
    xl2_ref[...] = xl2
    u2_ref[...] = (kf * dinv) * xl2


_tc2 = pl.pallas_call(
    _tc2_body,
    grid=(_GRID,),
    in_specs=[_rows_spec(2 * H), _rows_spec(2 * H), _rows_spec(2 * H),
              _rows_spec(1), _rows_spec(1), _full_spec(1, 2 * H),
              _full_spec(2 * H, H)],
    out_specs=[_rows_spec(H), _rows_spec(H)],
    out_shape=[jax.ShapeDtypeStruct((NP, H), _f32),
               jax.ShapeDtypeStruct((NP, H), _f32)],
)


def _tc3_body(a0_ref, a1_ref, xl2_ref, dinv_ref, kf_ref, b2_ref, wg_ref,
              asrc_ref, adst_ref,
              xaug_ref, asm_ref, ad_ref, ma_ref, md_ref):
    i = pl.program_id(0)
    dinv = dinv_ref[...]
    kf = kf_ref[...]
    h2 = jax.nn.relu(dinv * (a0_ref[...] + a1_ref[...])
                     + (dinv * dinv) * xl2_ref[...] + b2_ref[...])
    xl3 = jnp.dot(h2, wg_ref[...], preferred_element_type=_f32)
    as_ = jnp.dot(xl3, asrc_ref[...], preferred_element_type=_f32)
    ad_ = jnp.dot(xl3, adst_ref[...], preferred_element_type=_f32)
    asm = jnp.where(kf > 0, as_, -1e30)
    xaug_ref[...] = jnp.concatenate(
        [xl3, jnp.ones((_BLK, 1), _f32), jnp.zeros((_BLK, 15), _f32)], axis=1)
    asm_ref[...] = asm
    ad_ref[...] = ad_

    @pl.when(i == 0)
    def _():
        ma_ref[...] = jnp.full((1, 1), -1e30, _f32)
        md_ref[...] = jnp.full((1, 1), -1e30, _f32)

    ma_ref[...] = jnp.maximum(ma_ref[...], jnp.max(asm))
    md_ref[...] = jnp.maximum(md_ref[...], jnp.max(ad_))


_tc3 = pl.pallas_call(
    _tc3_body,
    grid=(_GRID,),
    in_specs=[_rows_spec(H), _rows_spec(H), _rows_spec(H),
              _rows_spec(1), _rows_spec(1), _full_spec(1, H),
              _full_spec(H, D), _full_spec(D, 1), _full_spec(D, 1)],
    out_specs=[_rows_spec(GC), _rows_spec(1), _rows_spec(1),
               _full_spec(1, 1), _full_spec(1, 1)],
    out_shape=[jax.ShapeDtypeStruct((NP, GC), _f32),
               jax.ShapeDtypeStruct((NP, 1), _f32),
               jax.ShapeDtypeStruct((NP, 1), _f32),
               jax.ShapeDtypeStruct((1, 1), _f32),
               jax.ShapeDtypeStruct((1, 1), _f32)],
)


def _tc4_body(g0_ref, g1_ref, bg_ref, wc_ref, bc_ref, out_ref):
    g = g0_ref[...] + g1_ref[...]
    s = g[:, D:D + 1]
    z = jax.nn.relu(g[:, :D] / (s + 1e-16) + bg_ref[...])
    xr = 1.0 / (1.0 + jnp.exp(-z))
    lg = jnp.dot(xr, wc_ref[...], preferred_element_type=_f32) + bc_ref[...]
    m = jnp.max(lg, axis=1, keepdims=True)
    e = lg - m
    out_ref[...] = e - jnp.log(jnp.sum(jnp.exp(e), axis=1, keepdims=True))


_tc4 = pl.pallas_call(
    _tc4_body,
    grid=(_GRID,),
    in_specs=[_rows_spec(GC), _rows_spec(GC), _full_spec(1, D),
              _full_spec(D, C), _full_spec(1, C)],
    out_specs=_rows_spec(C),
    out_shape=jax.ShapeDtypeStruct((NP, C), _f32),
)

_sc_spmm_deg = _make_sc_spmm(16, preload_idx=True)
_sc_spmm_128 = _make_sc_spmm(2 * H)
_sc_spmm_64 = _make_sc_spmm(H, preload_idx=True)


def kernel(x, edge_index, W1, b1, W2, b2, Wg, att_src, att_dst, bg, Wc, bc):
    with jax.ensure_compile_time_eval():
        perm = jax.random.permutation(jax.random.key(42), N)
        mask_nodes = perm[: int(0.15 * N)]
        keptf = jnp.ones((N,), _f32).at[mask_nodes].set(0.0)
        kf_p = jnp.zeros((NP, 1), _f32).at[:N, 0].set(keptf)
        ktab = jnp.concatenate([kf_p, jnp.zeros((NP, 15), _f32)], axis=1)
    x_p = jnp.zeros((NP, D), _f32).at[:N].set(x)
    ar = jnp.arange(EP - E, dtype=_i32)
    pad = jnp.stack([N + (ar % (NP - N)), ar % N])
    ei = jnp.concatenate([edge_index, pad], axis=1)
    edges3 = jnp.stack(
        [ei[0].reshape(NW * NSTEP, K), ei[1].reshape(NW * NSTEP, K)], axis=1)

    degp = _sc_spmm_deg(ktab, edges3)
    xl1 = _tc1a(x_p, kf_p, W1)
    u1, dinv = _tc1b(xl1, kf_p, degp[0], degp[1])
    agg1 = _sc_spmm_128(u1, edges3)
    xl2, u2 = _tc2(agg1[0], agg1[1], xl1, dinv, kf_p,
                   b1.reshape(1, 2 * H), W2)
    agg2 = _sc_spmm_64(u2, edges3)
    xaug, asm, ad, ma, md = _tc3(agg2[0], agg2[1], xl2, dinv, kf_p,
                                 b2.reshape(1, H), Wg,
                                 att_src.reshape(D, 1), att_dst.reshape(D, 1))
    mglob = jnp.maximum(ma[0, 0] + md[0, 0], 0.0)
    mvec = jnp.full((L,), mglob, _f32)
    gat = _sc_gat(asm.reshape(NP), ad.reshape(NP), mvec, edges3, xaug)
    out = _tc4(gat[0], gat[1], bg.reshape(1, D), Wc, bc.reshape(1, C))
    return out[:N]


# speedup vs baseline: 1.3193x; 1.3193x over previous
import jax
import jax.numpy as jnp
from jax import lax
from jax.experimental import pallas as pl
from jax.experimental.pallas import tpu as pltpu
from jax.experimental.pallas import tpu_sc as plsc

N = 10000
E = 320000
D = 128
H = 64
C = 40
NP = 10240
NC, NS, L = 2, 16, 16
NW = NC * NS
K = 128
NSTEP = 80
EPWP = NSTEP * K
EP = NW * EPWP
RPT = NP // NS
GC = D + 16

_f32 = jnp.float32
_i32 = jnp.int32
_mesh = plsc.VectorSubcoreMesh(core_axis_name="c", subcore_axis_name="s")
_CP = pltpu.CompilerParams(needs_layout_passes=False, use_tc_tiling_on_sc=False)


def _zero_rows(rows, cols):
    for r in range(K):
        for c in range(cols // L):
            rows[r, pl.ds(c * L, L)] = jnp.zeros((L,), _f32)


def _fill_acc(rows, acc, sid):
    def cp(i, _):
        pltpu.sync_copy(rows, acc.at[pl.ds(sid * RPT + i * K, K)])
        return 0

    lax.fori_loop(0, RPT // K, cp, 0)


def _writeout(acc, rows, out, cid, sid):
    def wo(i, _):
        off = sid * RPT + i * K
        pltpu.sync_copy(acc.at[pl.ds(off, K)], rows)
        pltpu.sync_copy(rows, out.at[cid].at[pl.ds(off, K)])
        return 0

    lax.fori_loop(0, RPT // K, wo, 0)


def _sc_edge_loop(tab, edges3, acc, e0, e1, b0, b1, gs0, gs1, ss0, ss1,
                  base, scale0=None, scale1=None):
    pltpu.sync_copy(edges3.at[base], e0)
    pltpu.async_copy(tab.at[e0.at[0]], b0, gs0)

    def body(jj, _):
        c1 = base + 2 * jj + 1
        pltpu.sync_copy(edges3.at[c1], e1)
        pltpu.async_copy(tab.at[e1.at[0]], b1, gs1)
        pltpu.make_async_copy(tab.at[e0.at[0]], b0, gs0).wait()
        if scale0 is not None:
            scale0(jj, b0)
        s0 = pltpu.async_copy(b0, acc.at[e0.at[1]], ss0, add=True)
        pltpu.make_async_copy(tab.at[e1.at[0]], b1, gs1).wait()
        if scale1 is not None:
            scale1(jj, b1)
        s1 = pltpu.async_copy(b1, acc.at[e1.at[1]], ss1, add=True)
        s0.wait()
        cn = base + jnp.minimum(2 * jj + 2, NSTEP - 1)
        pltpu.sync_copy(edges3.at[cn], e0)
        pltpu.async_copy(tab.at[e0.at[0]], b0, gs0)
        s1.wait()
        return 0

    lax.fori_loop(0, NSTEP // 2, body, 0)
    pltpu.make_async_copy(tab.at[e0.at[0]], b0, gs0).wait()


def _sc_edge_loop_pre(tab, ebig, acc, b0, b1, gs0, gs1, ss0, ss1, nstep):
    pltpu.async_copy(tab.at[ebig.at[0, 0]], b0, gs0)

    def body(jj, _):
        c1 = 2 * jj + 1
        pltpu.async_copy(tab.at[ebig.at[c1, 0]], b1, gs1)
        pltpu.make_async_copy(tab.at[ebig.at[0, 0]], b0, gs0).wait()
        s0 = pltpu.async_copy(b0, acc.at[ebig.at[2 * jj, 1]], ss0, add=True)
        pltpu.make_async_copy(tab.at[ebig.at[0, 0]], b1, gs1).wait()
        s1 = pltpu.async_copy(b1, acc.at[ebig.at[c1, 1]], ss1, add=True)
        s0.wait()
        cn = jnp.minimum(2 * jj + 2, nstep - 1)
        pltpu.async_copy(tab.at[ebig.at[cn, 0]], b0, gs0)
        s1.wait()
        return 0

    lax.fori_loop(0, nstep // 2, body, 0)
    pltpu.make_async_copy(tab.at[ebig.at[0, 0]], b0, gs0).wait()


def _make_sc_spmm(cols, preload_idx=False):
    if preload_idx:
        nphase = 1 if cols <= H else 2
        pstep = NSTEP // nphase

        def body(tab, edges3, out, ebig, b0, b1, acc, gs0, gs1, ss0, ss1):
            cid = lax.axis_index("c")
            sid = lax.axis_index("s")
            base = (cid * NS + sid) * NSTEP
            _zero_rows(b0, cols)
            _fill_acc(b0, acc, sid)
            plsc.subcore_barrier()
            for ph in range(nphase):
                pltpu.sync_copy(
                    edges3.at[pl.ds(base + ph * pstep, pstep)], ebig)
                _sc_edge_loop_pre(tab, ebig, acc, b0, b1,
                                  gs0, gs1, ss0, ss1, pstep)
            plsc.subcore_barrier()
            _writeout(acc, b0, out, cid, sid)

        idx_scratch = [pltpu.VMEM((NSTEP // nphase, 2, K), _i32)]
    else:
        def body(tab, edges3, out, e0, e1, b0, b1, acc, gs0, gs1, ss0, ss1):
            cid = lax.axis_index("c")
            sid = lax.axis_index("s")
            _zero_rows(b0, cols)
            _fill_acc(b0, acc, sid)
            plsc.subcore_barrier()
            base = (cid * NS + sid) * NSTEP
            _sc_edge_loop(tab, edges3, acc, e0, e1, b0, b1,
                          gs0, gs1, ss0, ss1, base)
            plsc.subcore_barrier()
            _writeout(acc, b0, out, cid, sid)

        idx_scratch = [pltpu.VMEM((2, K), _i32), pltpu.VMEM((2, K), _i32)]

    return pl.kernel(
        body,
        out_type=jax.ShapeDtypeStruct((NC, NP, cols), _f32),
        mesh=_mesh,
        compiler_params=_CP,
        scratch_types=idx_scratch + [
            pltpu.VMEM((K, cols), _f32),
            pltpu.VMEM((K, cols), _f32),
            pltpu.VMEM_SHARED((NP, cols), _f32),
            pltpu.SemaphoreType.DMA,
            pltpu.SemaphoreType.DMA,
            pltpu.SemaphoreType.DMA,
            pltpu.SemaphoreType.DMA,
        ],
    )


def _sc_gat_body(asm, ad, mv, edges3, xaug, out,
                 e0, e1, av0, av1, dv0, dv1, b0, b1, mvv, acc,
                 gs0, gs1, as0, as1, ds0, ds1, ss0, ss1):
    cid = lax.axis_index("c")
    sid = lax.axis_index("s")
    _zero_rows(b0, GC)
    _fill_acc(b0, acc, sid)
    pltpu.sync_copy(mv, mvv)
    plsc.subcore_barrier()
    base = (cid * NS + sid) * NSTEP
    m = mvv[...]

    def issue(c, e, b, av, dv, gs, asem, dsem):
        pltpu.sync_copy(edges3.at[c], e)
        pltpu.async_copy(xaug.at[e.at[0]], b, gs)
        pltpu.async_copy(asm.at[e.at[0]], av, asem)
        pltpu.async_copy(ad.at[e.at[1]], dv, dsem)

    def tcomp(e, b, av, dv, gs, asem, dsem):
        pltpu.make_async_copy(asm.at[e.at[0]], av, asem).wait()
        pltpu.make_async_copy(ad.at[e.at[1]], dv, dsem).wait()
        for i in range(K // L):
            v = av[pl.ds(i * L, L)] + dv[pl.ds(i * L, L)]
            av[pl.ds(i * L, L)] = jnp.exp(jnp.maximum(v, 0.2 * v) - m)
        pltpu.make_async_copy(xaug.at[e.at[0]], b, gs).wait()

        def rowm(r, _):
            t = plsc.load_gather(av, [jnp.full((L,), r, _i32)])
            for c in range(GC // L):
                b[r, pl.ds(c * L, L)] = b[r, pl.ds(c * L, L)] * t
            return 0

        lax.fori_loop(0, K, rowm, 0)

    issue(base, e0, b0, av0, dv0, gs0, as0, ds0)

    def body(jj, _):
        issue(base + 2 * jj + 1, e1, b1, av1, dv1, gs1, as1, ds1)
        tcomp(e0, b0, av0, dv0, gs0, as0, ds0)
        s0 = pltpu.async_copy(b0, acc.at[e0.at[1]], ss0, add=True)
        tcomp(e1, b1, av1, dv1, gs1, as1, ds1)
        s1 = pltpu.async_copy(b1, acc.at[e1.at[1]], ss1, add=True)
        s0.wait()
        cn = base + jnp.minimum(2 * jj + 2, NSTEP - 1)
        issue(cn, e0, b0, av0, dv0, gs0, as0, ds0)
        s1.wait()
        return 0

    lax.fori_loop(0, NSTEP // 2, body, 0)
    pltpu.make_async_copy(xaug.at[e0.at[0]], b0, gs0).wait()
    pltpu.make_async_copy(asm.at[e0.at[0]], av0, as0).wait()
    pltpu.make_async_copy(ad.at[e0.at[1]], dv0, ds0).wait()
    plsc.subcore_barrier()
    _writeout(acc, b0, out, cid, sid)


_sc_gat = pl.kernel(
    _sc_gat_body,
    out_type=jax.ShapeDtypeStruct((NC, NP, GC), _f32),
    mesh=_mesh,
    compiler_params=_CP,
    scratch_types=[
        pltpu.VMEM((2, K), _i32),
        pltpu.VMEM((2, K), _i32),
        pltpu.VMEM((K,), _f32),
        pltpu.VMEM((K,), _f32),
        pltpu.VMEM((K,), _f32),
        pltpu.VMEM((K,), _f32),
        pltpu.VMEM((K, GC), _f32),
        pltpu.VMEM((K, GC), _f32),
        pltpu.VMEM((L,), _f32),
        pltpu.VMEM_SHARED((NP, GC), _f32),
        pltpu.SemaphoreType.DMA,
        pltpu.SemaphoreType.DMA,
        pltpu.SemaphoreType.DMA,
        pltpu.SemaphoreType.DMA,
        pltpu.SemaphoreType.DMA,
        pltpu.SemaphoreType.DMA,
        pltpu.SemaphoreType.DMA,
        pltpu.SemaphoreType.DMA,
    ],
)


_BLK = 2048
_GRID = NP // _BLK


def _rows_spec(cols):
    return pl.BlockSpec((_BLK, cols), lambda i: (i, 0))


def _full_spec(r, c):
    return pl.BlockSpec((r, c), lambda i: (0, 0))


def _tc1a_body(x_ref, kf_ref, w1_ref, xl1_ref):
    xl1_ref[...] = jnp.dot(kf_ref[...] * x_ref[...], w1_ref[...],
                           preferred_element_type=_f32)


_tc1a = pl.pallas_call(
    _tc1a_body,
    grid=(_GRID,),
    in_specs=[_rows_spec(D), _rows_spec(1), _full_spec(D, 2 * H)],
    out_specs=_rows_spec(2 * H),
    out_shape=jax.ShapeDtypeStruct((NP, 2 * H), _f32),
)


def _tc1b_body(xl1_ref, kf_ref, d0_ref, d1_ref, u1_ref, dinv_ref):
    kf = kf_ref[...]
    deg = d0_ref[...][:, 0:1] + d1_ref[...][:, 0:1] + 1.0
    dinv = lax.rsqrt(deg)
    u1_ref[...] = (kf * dinv) * xl1_ref[...]
    dinv_ref[...] = dinv


_tc1b = pl.pallas_call(
    _tc1b_body,
    grid=(_GRID,),
    in_specs=[_rows_spec(2 * H), _rows_spec(1), _rows_spec(16),
              _rows_spec(16)],
    out_specs=[_rows_spec(2 * H), _rows_spec(1)],
    out_shape=[jax.ShapeDtypeStruct((NP, 2 * H), _f32),
               jax.ShapeDtypeStruct((NP, 1), _f32)],
)


def _tc2_body(a0_ref, a1_ref, xl1_ref, dinv_ref, kf_ref, b1_ref, w2_ref,
              xl2_ref, u2_ref):
    dinv = dinv_ref[...]
    kf = kf_ref[...]
    h1 = jax.nn.relu(dinv * (a0_ref[...] + a1_ref[...])
                     + (dinv * dinv) * xl1_ref[...] + b1_ref[...])
    xl2 = jnp.dot(h1, w2_ref[...], preferred_element_type=_f32)
    xl2_ref[...] = xl2
    u2_ref[...] = (kf * dinv) * xl2


_tc2 = pl.pallas_call(
    _tc2_body,
    grid=(_GRID,),
    in_specs=[_rows_spec(2 * H), _rows_spec(2 * H), _rows_spec(2 * H),
              _rows_spec(1), _rows_spec(1), _full_spec(1, 2 * H),
              _full_spec(2 * H, H)],
    out_specs=[_rows_spec(H), _rows_spec(H)],
    out_shape=[jax.ShapeDtypeStruct((NP, H), _f32),
               jax.ShapeDtypeStruct((NP, H), _f32)],
)


def _tc3_body(a0_ref, a1_ref, xl2_ref, dinv_ref, kf_ref, b2_ref, wg_ref,
              asrc_ref, adst_ref,
              xaug_ref, asm_ref, ad_ref, ma_ref, md_ref):
    i = pl.program_id(0)
    dinv = dinv_ref[...]
    kf = kf_ref[...]
    h2 = jax.nn.relu(dinv * (a0_ref[...] + a1_ref[...])
                     + (dinv * dinv) * xl2_ref[...] + b2_ref[...])
    xl3 = jnp.dot(h2, wg_ref[...], preferred_element_type=_f32)
    as_ = jnp.dot(xl3, asrc_ref[...], preferred_element_type=_f32)
    ad_ = jnp.dot(xl3, adst_ref[...], preferred_element_type=_f32)
    asm = jnp.where(kf > 0, as_, -1e30)
    xaug_ref[...] = jnp.concatenate(
        [xl3, jnp.ones((_BLK, 1), _f32), jnp.zeros((_BLK, 15), _f32)], axis=1)
    asm_ref[...] = asm
    ad_ref[...] = ad_

    @pl.when(i == 0)
    def _():
        ma_ref[...] = jnp.full((1, 1), -1e30, _f32)
        md_ref[...] = jnp.full((1, 1), -1e30, _f32)

    ma_ref[...] = jnp.maximum(ma_ref[...], jnp.max(asm))
    md_ref[...] = jnp.maximum(md_ref[...], jnp.max(ad_))


_tc3 = pl.pallas_call(
    _tc3_body,
    grid=(_GRID,),
    in_specs=[_rows_spec(H), _rows_spec(H), _rows_spec(H),
              _rows_spec(1), _rows_spec(1), _full_spec(1, H),
              _full_spec(H, D), _full_spec(D, 1), _full_spec(D, 1)],
    out_specs=[_rows_spec(GC), _rows_spec(1), _rows_spec(1),
               _full_spec(1, 1), _full_spec(1, 1)],
    out_shape=[jax.ShapeDtypeStruct((NP, GC), _f32),
               jax.ShapeDtypeStruct((NP, 1), _f32),
               jax.ShapeDtypeStruct((NP, 1), _f32),
               jax.ShapeDtypeStruct((1, 1), _f32),
               jax.ShapeDtypeStruct((1, 1), _f32)],
)


def _tc4_body(g0_ref, g1_ref, bg_ref, wc_ref, bc_ref, out_ref):
    g = g0_ref[...] + g1_ref[...]
    s = g[:, D:D + 1]
    z = jax.nn.relu(g[:, :D] / (s + 1e-16) + bg_ref[...])
    xr = 1.0 / (1.0 + jnp.exp(-z))
    lg = jnp.dot(xr, wc_ref[...], preferred_element_type=_f32) + bc_ref[...]
    m = jnp.max(lg, axis=1, keepdims=True)
    e = lg - m
    out_ref[...] = e - jnp.log(jnp.sum(jnp.exp(e), axis=1, keepdims=True))


_tc4 = pl.pallas_call(
    _tc4_body,
    grid=(_GRID,),
    in_specs=[_rows_spec(GC), _rows_spec(GC), _full_spec(1, D),
              _full_spec(D, C), _full_spec(1, C)],
    out_specs=_rows_spec(C),
    out_shape=jax.ShapeDtypeStruct((NP, C), _f32),
)

_sc_spmm_deg = _make_sc_spmm(16, preload_idx=True)
_sc_spmm_128 = _make_sc_spmm(2 * H)
_sc_spmm_64 = _make_sc_spmm(H, preload_idx=True)


def kernel(x, edge_index, W1, b1, W2, b2, Wg, att_src, att_dst, bg, Wc, bc):
    with jax.ensure_compile_time_eval():
        perm = jax.random.permutation(jax.random.key(42), N)
        mask_nodes = perm[: int(0.15 * N)]
        keptf = jnp.ones((N,), _f32).at[mask_nodes].set(0.0)
        kf_p = jnp.zeros((NP, 1), _f32).at[:N, 0].set(keptf)
        ktab = jnp.concatenate([kf_p, jnp.zeros((NP, 15), _f32)], axis=1)
    x_p = jnp.zeros((NP, D), _f32).at[:N].set(x)
    ar = jnp.arange(EP - E, dtype=_i32)
    pad = jnp.stack([N + (ar % (NP - N)), ar % N])
    ei = jnp.concatenate([edge_index, pad], axis=1)
    edges3 = jnp.stack(
        [ei[0].reshape(NW * NSTEP, K), ei[1].reshape(NW * NSTEP, K)], axis=1)

    degp = _sc_spmm_deg(ktab, edges3)
    xl1 = _tc1a(x_p, kf_p, W1)
    u1, dinv = _tc1b(xl1, kf_p, degp[0], degp[1])
    agg1 = _sc_spmm_128(u1, edges3)
    xl2, u2 = _tc2(agg1[0], agg1[1], xl1, dinv, kf_p,
                   b1.reshape(1, 2 * H), W2)
    agg2 = _sc_spmm_64(u2, edges3)
    xaug, asm, ad, ma, md = _tc3(agg2[0], agg2[1], xl2, dinv, kf_p,
                                 b2.reshape(1, H), Wg,
                                 att_src.reshape(D, 1), att_dst.reshape(D, 1))
    mglob = jnp.maximum(ma[0, 0] + md[0, 0], 0.0)
    mvec = jnp.full((L,), mglob, _f32)
    gat = _sc_gat(asm.reshape(NP), ad.reshape(NP), mvec, edges3, xaug)
    out = _tc4(gat[0], gat[1], bg.reshape(1, D), Wc, bc.reshape(1, C))
    return out[:N]


# speedup vs baseline: 1.3720x; 1.0400x over previous
import jax
import jax.numpy as jnp
from jax import lax
from jax.experimental import pallas as pl
from jax.experimental.pallas import tpu as pltpu
from jax.experimental.pallas import tpu_sc as plsc

N = 10000
E = 320000
D = 128
H = 64
C = 40
NP = 10240
NC, NS, L = 2, 16, 16
NW = NC * NS
K = 128
NSTEP = 80
EPWP = NSTEP * K
EP = NW * EPWP
RPT = NP // NS
GC = D + 16

_f32 = jnp.float32
_i32 = jnp.int32
_mesh = plsc.VectorSubcoreMesh(core_axis_name="c", subcore_axis_name="s")
_CP = pltpu.CompilerParams(needs_layout_passes=False, use_tc_tiling_on_sc=False)


def _zero_rows(rows, cols):
    for r in range(K):
        for c in range(cols // L):
            rows[r, pl.ds(c * L, L)] = jnp.zeros((L,), _f32)


def _fill_acc(rows, acc, sid):
    def cp(i, _):
        pltpu.sync_copy(rows, acc.at[pl.ds(sid * RPT + i * K, K)])
        return 0

    lax.fori_loop(0, RPT // K, cp, 0)


def _writeout(acc, rows, out, cid, sid):
    def wo(i, _):
        off = sid * RPT + i * K
        pltpu.sync_copy(acc.at[pl.ds(off, K)], rows)
        pltpu.sync_copy(rows, out.at[cid].at[pl.ds(off, K)])
        return 0

    lax.fori_loop(0, RPT // K, wo, 0)


def _sc_edge_loop(tab, edges3, acc, e0, e1, b0, b1, gs0, gs1, ss0, ss1,
                  base, scale0=None, scale1=None):
    pltpu.sync_copy(edges3.at[base], e0)
    pltpu.async_copy(tab.at[e0.at[0]], b0, gs0)

    def body(jj, _):
        c1 = base + 2 * jj + 1
        pltpu.sync_copy(edges3.at[c1], e1)
        pltpu.async_copy(tab.at[e1.at[0]], b1, gs1)
        pltpu.make_async_copy(tab.at[e0.at[0]], b0, gs0).wait()
        if scale0 is not None:
            scale0(jj, b0)
        s0 = pltpu.async_copy(b0, acc.at[e0.at[1]], ss0, add=True)
        pltpu.make_async_copy(tab.at[e1.at[0]], b1, gs1).wait()
        if scale1 is not None:
            scale1(jj, b1)
        s1 = pltpu.async_copy(b1, acc.at[e1.at[1]], ss1, add=True)
        s0.wait()
        cn = base + jnp.minimum(2 * jj + 2, NSTEP - 1)
        pltpu.sync_copy(edges3.at[cn], e0)
        pltpu.async_copy(tab.at[e0.at[0]], b0, gs0)
        s1.wait()
        return 0

    lax.fori_loop(0, NSTEP // 2, body, 0)
    pltpu.make_async_copy(tab.at[e0.at[0]], b0, gs0).wait()


def _sc_edge_loop_pre(tab, ebig, acc, b0, b1, gs0, gs1, ss0, ss1, nstep):
    pltpu.async_copy(tab.at[ebig.at[0, 0]], b0, gs0)

    def body(jj, _):
        c1 = 2 * jj + 1
        pltpu.async_copy(tab.at[ebig.at[c1, 0]], b1, gs1)
        pltpu.make_async_copy(tab.at[ebig.at[0, 0]], b0, gs0).wait()
        s0 = pltpu.async_copy(b0, acc.at[ebig.at[2 * jj, 1]], ss0, add=True)
        pltpu.make_async_copy(tab.at[ebig.at[0, 0]], b1, gs1).wait()
        s1 = pltpu.async_copy(b1, acc.at[ebig.at[c1, 1]], ss1, add=True)
        s0.wait()
        cn = jnp.minimum(2 * jj + 2, nstep - 1)
        pltpu.async_copy(tab.at[ebig.at[cn, 0]], b0, gs0)
        s1.wait()
        return 0

    lax.fori_loop(0, nstep // 2, body, 0)
    pltpu.make_async_copy(tab.at[ebig.at[0, 0]], b0, gs0).wait()


def _make_sc_spmm(cols, preload_idx=False):
    if preload_idx:
        nphase = 1 if cols <= H else 2
        pstep = NSTEP // nphase

        def body(tab, edges3, out, ebig, b0, b1, acc, gs0, gs1, ss0, ss1):
            cid = lax.axis_index("c")
            sid = lax.axis_index("s")
            base = (cid * NS + sid) * NSTEP
            _zero_rows(b0, cols)
            _fill_acc(b0, acc, sid)
            plsc.subcore_barrier()
            for ph in range(nphase):
                pltpu.sync_copy(
                    edges3.at[pl.ds(base + ph * pstep, pstep)], ebig)
                _sc_edge_loop_pre(tab, ebig, acc, b0, b1,
                                  gs0, gs1, ss0, ss1, pstep)
            plsc.subcore_barrier()
            _writeout(acc, b0, out, cid, sid)

        idx_scratch = [pltpu.VMEM((NSTEP // nphase, 2, K), _i32)]
    else:
        def body(tab, edges3, out, e0, e1, b0, b1, acc, gs0, gs1, ss0, ss1):
            cid = lax.axis_index("c")
            sid = lax.axis_index("s")
            _zero_rows(b0, cols)
            _fill_acc(b0, acc, sid)
            plsc.subcore_barrier()
            base = (cid * NS + sid) * NSTEP
            _sc_edge_loop(tab, edges3, acc, e0, e1, b0, b1,
                          gs0, gs1, ss0, ss1, base)
            plsc.subcore_barrier()
            _writeout(acc, b0, out, cid, sid)

        idx_scratch = [pltpu.VMEM((2, K), _i32), pltpu.VMEM((2, K), _i32)]

    return pl.kernel(
        body,
        out_type=jax.ShapeDtypeStruct((NC, NP, cols), _f32),
        mesh=_mesh,
        compiler_params=_CP,
        scratch_types=idx_scratch + [
            pltpu.VMEM((K, cols), _f32),
            pltpu.VMEM((K, cols), _f32),
            pltpu.VMEM_SHARED((NP, cols), _f32),
            pltpu.SemaphoreType.DMA,
            pltpu.SemaphoreType.DMA,
            pltpu.SemaphoreType.DMA,
            pltpu.SemaphoreType.DMA,
        ],
    )


def _sc_gat_body(asm, ad, mv, edges3, xaug, out,
                 e0, e1, av0, av1, dv0, dv1, b0, b1, mvv, acc,
                 gs0, gs1, as0, as1, ds0, ds1, ss0, ss1):
    cid = lax.axis_index("c")
    sid = lax.axis_index("s")
    _zero_rows(b0, GC)
    _fill_acc(b0, acc, sid)
    pltpu.sync_copy(mv, mvv)
    plsc.subcore_barrier()
    base = (cid * NS + sid) * NSTEP
    m = mvv[...]

    def issue(c, e, b, av, dv, gs, asem, dsem):
        pltpu.sync_copy(edges3.at[c], e)
        pltpu.async_copy(xaug.at[e.at[0]], b, gs)
        pltpu.async_copy(asm.at[e.at[0]], av, asem)
        pltpu.async_copy(ad.at[e.at[1]], dv, dsem)

    def tcomp(e, b, av, dv, gs, asem, dsem):
        pltpu.make_async_copy(asm.at[e.at[0]], av, asem).wait()
        pltpu.make_async_copy(ad.at[e.at[1]], dv, dsem).wait()
        for i in range(K // L):
            v = av[pl.ds(i * L, L)] + dv[pl.ds(i * L, L)]
            av[pl.ds(i * L, L)] = jnp.exp(jnp.maximum(v, 0.2 * v) - m)
        pltpu.make_async_copy(xaug.at[e.at[0]], b, gs).wait()

        @plsc.parallel_loop(0, K, unroll=2)
        def _(r):
            t = plsc.load_gather(av, [jnp.full((L,), r, _i32)])
            for c in range(GC // L):
                b[r, pl.ds(c * L, L)] = b[r, pl.ds(c * L, L)] * t

    issue(base, e0, b0, av0, dv0, gs0, as0, ds0)

    def body(jj, _):
        issue(base + 2 * jj + 1, e1, b1, av1, dv1, gs1, as1, ds1)
        tcomp(e0, b0, av0, dv0, gs0, as0, ds0)
        s0 = pltpu.async_copy(b0, acc.at[e0.at[1]], ss0, add=True)
        tcomp(e1, b1, av1, dv1, gs1, as1, ds1)
        s1 = pltpu.async_copy(b1, acc.at[e1.at[1]], ss1, add=True)
        s0.wait()
        cn = base + jnp.minimum(2 * jj + 2, NSTEP - 1)
        issue(cn, e0, b0, av0, dv0, gs0, as0, ds0)
        s1.wait()
        return 0

    lax.fori_loop(0, NSTEP // 2, body, 0)
    pltpu.make_async_copy(xaug.at[e0.at[0]], b0, gs0).wait()
    pltpu.make_async_copy(asm.at[e0.at[0]], av0, as0).wait()
    pltpu.make_async_copy(ad.at[e0.at[1]], dv0, ds0).wait()
    plsc.subcore_barrier()
    _writeout(acc, b0, out, cid, sid)


_sc_gat = pl.kernel(
    _sc_gat_body,
    out_type=jax.ShapeDtypeStruct((NC, NP, GC), _f32),
    mesh=_mesh,
    compiler_params=_CP,
    scratch_types=[
        pltpu.VMEM((2, K), _i32),
        pltpu.VMEM((2, K), _i32),
        pltpu.VMEM((K,), _f32),
        pltpu.VMEM((K,), _f32),
        pltpu.VMEM((K,), _f32),
        pltpu.VMEM((K,), _f32),
        pltpu.VMEM((K, GC), _f32),
        pltpu.VMEM((K, GC), _f32),
        pltpu.VMEM((L,), _f32),
        pltpu.VMEM_SHARED((NP, GC), _f32),
        pltpu.SemaphoreType.DMA,
        pltpu.SemaphoreType.DMA,
        pltpu.SemaphoreType.DMA,
        pltpu.SemaphoreType.DMA,
        pltpu.SemaphoreType.DMA,
        pltpu.SemaphoreType.DMA,
        pltpu.SemaphoreType.DMA,
        pltpu.SemaphoreType.DMA,
    ],
)


_BLK = 2048
_GRID = NP // _BLK


def _rows_spec(cols):
    return pl.BlockSpec((_BLK, cols), lambda i: (i, 0))


def _full_spec(r, c):
    return pl.BlockSpec((r, c), lambda i: (0, 0))


def _tc1a_body(x_ref, kf_ref, w1_ref, xl1_ref):
    xl1_ref[...] = jnp.dot(kf_ref[...] * x_ref[...], w1_ref[...],
                           preferred_element_type=_f32)


_tc1a = pl.pallas_call(
    _tc1a_body,
    grid=(_GRID,),
    in_specs=[_rows_spec(D), _rows_spec(1), _full_spec(D, 2 * H)],
    out_specs=_rows_spec(2 * H),
    out_shape=jax.ShapeDtypeStruct((NP, 2 * H), _f32),
)


def _tc1b_body(xl1_ref, kf_ref, d0_ref, d1_ref, u1_ref, dinv_ref):
    kf = kf_ref[...]
    deg = d0_ref[...][:, 0:1] + d1_ref[...][:, 0:1] + 1.0
    dinv = lax.rsqrt(deg)
    u1_ref[...] = (kf * dinv) * xl1_ref[...]
    dinv_ref[...] = dinv


_tc1b = pl.pallas_call(
    _tc1b_body,
    grid=(_GRID,),
    in_specs=[_rows_spec(2 * H), _rows_spec(1), _rows_spec(16),
              _rows_spec(16)],
    out_specs=[_rows_spec(2 * H), _rows_spec(1)],
    out_shape=[jax.ShapeDtypeStruct((NP, 2 * H), _f32),
               jax.ShapeDtypeStruct((NP, 1), _f32)],
)


def _tc2_body(a0_ref, a1_ref, xl1_ref, dinv_ref, kf_ref, b1_ref, w2_ref,
              xl2_ref, u2_ref):
    dinv = dinv_ref[...]
    kf = kf_ref[...]
    h1 = jax.nn.relu(dinv * (a0_ref[...] + a1_ref[...])
                     + (dinv * dinv) * xl1_ref[...] + b1_ref[...])
    xl2 = jnp.dot(h1, w2_ref[...], preferred_element_type=_f32)
    xl2_ref[...] = xl2
    u2_ref[...] = (kf * dinv) * xl2


_tc2 = pl.pallas_call(
    _tc2_body,
    grid=(_GRID,),
    in_specs=[_rows_spec(2 * H), _rows_spec(2 * H), _rows_spec(2 * H),
              _rows_spec(1), _rows_spec(1), _full_spec(1, 2 * H),
              _full_spec(2 * H, H)],
    out_specs=[_rows_spec(H), _rows_spec(H)],
    out_shape=[jax.ShapeDtypeStruct((NP, H), _f32),
               jax.ShapeDtypeStruct((NP, H), _f32)],
)


def _tc3_body(a0_ref, a1_ref, xl2_ref, dinv_ref, kf_ref, b2_ref, wg_ref,
              asrc_ref, adst_ref,
              xaug_ref, asm_ref, ad_ref, ma_ref, md_ref):
    i = pl.program_id(0)
    dinv = dinv_ref[...]
    kf = kf_ref[...]
    h2 = jax.nn.relu(dinv * (a0_ref[...] + a1_ref[...])
                     + (dinv * dinv) * xl2_ref[...] + b2_ref[...])
    xl3 = jnp.dot(h2, wg_ref[...], preferred_element_type=_f32)
    as_ = jnp.dot(xl3, asrc_ref[...], preferred_element_type=_f32)
    ad_ = jnp.dot(xl3, adst_ref[...], preferred_element_type=_f32)
    asm = jnp.where(kf > 0, as_, -1e30)
    xaug_ref[...] = jnp.concatenate(
        [xl3, jnp.ones((_BLK, 1), _f32), jnp.zeros((_BLK, 15), _f32)], axis=1)
    asm_ref[...] = asm
    ad_ref[...] = ad_

    @pl.when(i == 0)
    def _():
        ma_ref[...] = jnp.full((1, 1), -1e30, _f32)
        md_ref[...] = jnp.full((1, 1), -1e30, _f32)

    ma_ref[...] = jnp.maximum(ma_ref[...], jnp.max(asm))
    md_ref[...] = jnp.maximum(md_ref[...], jnp.max(ad_))


_tc3 = pl.pallas_call(
    _tc3_body,
    grid=(_GRID,),
    in_specs=[_rows_spec(H), _rows_spec(H), _rows_spec(H),
              _rows_spec(1), _rows_spec(1), _full_spec(1, H),
              _full_spec(H, D), _full_spec(D, 1), _full_spec(D, 1)],
    out_specs=[_rows_spec(GC), _rows_spec(1), _rows_spec(1),
               _full_spec(1, 1), _full_spec(1, 1)],
    out_shape=[jax.ShapeDtypeStruct((NP, GC), _f32),
               jax.ShapeDtypeStruct((NP, 1), _f32),
               jax.ShapeDtypeStruct((NP, 1), _f32),
               jax.ShapeDtypeStruct((1, 1), _f32),
               jax.ShapeDtypeStruct((1, 1), _f32)],
)


def _tc4_body(g0_ref, g1_ref, bg_ref, wc_ref, bc_ref, out_ref):
    g = g0_ref[...] + g1_ref[...]
    s = g[:, D:D + 1]
    z = jax.nn.relu(g[:, :D] / (s + 1e-16) + bg_ref[...])
    xr = 1.0 / (1.0 + jnp.exp(-z))
    lg = jnp.dot(xr, wc_ref[...], preferred_element_type=_f32) + bc_ref[...]
    m = jnp.max(lg, axis=1, keepdims=True)
    e = lg - m
    out_ref[...] = e - jnp.log(jnp.sum(jnp.exp(e), axis=1, keepdims=True))


_tc4 = pl.pallas_call(
    _tc4_body,
    grid=(_GRID,),
    in_specs=[_rows_spec(GC), _rows_spec(GC), _full_spec(1, D),
              _full_spec(D, C), _full_spec(1, C)],
    out_specs=_rows_spec(C),
    out_shape=jax.ShapeDtypeStruct((NP, C), _f32),
)

_sc_spmm_deg = _make_sc_spmm(16, preload_idx=True)
_sc_spmm_128 = _make_sc_spmm(2 * H)
_sc_spmm_64 = _make_sc_spmm(H, preload_idx=True)


def kernel(x, edge_index, W1, b1, W2, b2, Wg, att_src, att_dst, bg, Wc, bc):
    with jax.ensure_compile_time_eval():
        perm = jax.random.permutation(jax.random.key(42), N)
        mask_nodes = perm[: int(0.15 * N)]
        keptf = jnp.ones((N,), _f32).at[mask_nodes].set(0.0)
        kf_p = jnp.zeros((NP, 1), _f32).at[:N, 0].set(keptf)
        ktab = jnp.concatenate([kf_p, jnp.zeros((NP, 15), _f32)], axis=1)
    x_p = jnp.zeros((NP, D), _f32).at[:N].set(x)
    ar = jnp.arange(EP - E, dtype=_i32)
    pad = jnp.stack([N + (ar % (NP - N)), ar % N])
    ei = jnp.concatenate([edge_index, pad], axis=1)
    edges3 = jnp.stack(
        [ei[0].reshape(NW * NSTEP, K), ei[1].reshape(NW * NSTEP, K)], axis=1)

    degp = _sc_spmm_deg(ktab, edges3)
    xl1 = _tc1a(x_p, kf_p, W1)
    u1, dinv = _tc1b(xl1, kf_p, degp[0], degp[1])
    agg1 = _sc_spmm_128(u1, edges3)
    xl2, u2 = _tc2(agg1[0], agg1[1], xl1, dinv, kf_p,
                   b1.reshape(1, 2 * H), W2)
    agg2 = _sc_spmm_64(u2, edges3)
    xaug, asm, ad, ma, md = _tc3(agg2[0], agg2[1], xl2, dinv, kf_p,
                                 b2.reshape(1, H), Wg,
                                 att_src.reshape(D, 1), att_dst.reshape(D, 1))
    mglob = jnp.maximum(ma[0, 0] + md[0, 0], 0.0)
    mvec = jnp.full((L,), mglob, _f32)
    gat = _sc_gat(asm.reshape(NP), ad.reshape(NP), mvec, edges3, xaug)
    out = _tc4(gat[0], gat[1], bg.reshape(1, D), Wc, bc.reshape(1, C))
    return out[:N]


# speedup vs baseline: 1.3731x; 1.0008x over previous
import jax
import jax.numpy as jnp
from jax import lax
from jax.experimental import pallas as pl
from jax.experimental.pallas import tpu as pltpu
from jax.experimental.pallas import tpu_sc as plsc

N = 10000
E = 320000
D = 128
H = 64
C = 40
NP = 10240
NC, NS, L = 2, 16, 16
NW = NC * NS
K = 128
NSTEP = 80
EPWP = NSTEP * K
EP = NW * EPWP
RPT = NP // NS
GC = D + 16

_f32 = jnp.float32
_i32 = jnp.int32
_mesh = plsc.VectorSubcoreMesh(core_axis_name="c", subcore_axis_name="s")
_CP = pltpu.CompilerParams(needs_layout_passes=False, use_tc_tiling_on_sc=False)


def _zero_rows(rows, cols):
    for r in range(K):
        for c in range(cols // L):
            rows[r, pl.ds(c * L, L)] = jnp.zeros((L,), _f32)


def _fill_acc(rows, acc, sid):
    def cp(i, _):
        pltpu.sync_copy(rows, acc.at[pl.ds(sid * RPT + i * K, K)])
        return 0

    lax.fori_loop(0, RPT // K, cp, 0)


def _writeout(acc, rows, out, cid, sid):
    def wo(i, _):
        off = sid * RPT + i * K
        pltpu.sync_copy(acc.at[pl.ds(off, K)], rows)
        pltpu.sync_copy(rows, out.at[cid].at[pl.ds(off, K)])
        return 0

    lax.fori_loop(0, RPT // K, wo, 0)


def _sc_edge_loop(tab, edges3, acc, e0, e1, b0, b1, gs0, gs1, ss0, ss1,
                  base, scale0=None, scale1=None):
    pltpu.sync_copy(edges3.at[base], e0)
    pltpu.async_copy(tab.at[e0.at[0]], b0, gs0)

    def body(jj, _):
        c1 = base + 2 * jj + 1
        pltpu.sync_copy(edges3.at[c1], e1)
        pltpu.async_copy(tab.at[e1.at[0]], b1, gs1)
        pltpu.make_async_copy(tab.at[e0.at[0]], b0, gs0).wait()
        if scale0 is not None:
            scale0(jj, b0)
        s0 = pltpu.async_copy(b0, acc.at[e0.at[1]], ss0, add=True)
        pltpu.make_async_copy(tab.at[e1.at[0]], b1, gs1).wait()
        if scale1 is not None:
            scale1(jj, b1)
        s1 = pltpu.async_copy(b1, acc.at[e1.at[1]], ss1, add=True)
        s0.wait()
        cn = base + jnp.minimum(2 * jj + 2, NSTEP - 1)
        pltpu.sync_copy(edges3.at[cn], e0)
        pltpu.async_copy(tab.at[e0.at[0]], b0, gs0)
        s1.wait()
        return 0

    lax.fori_loop(0, NSTEP // 2, body, 0)
    pltpu.make_async_copy(tab.at[e0.at[0]], b0, gs0).wait()


def _sc_edge_loop_pre(tab, ebig, acc, b0, b1, gs0, gs1, ss0, ss1, nstep):
    pltpu.async_copy(tab.at[ebig.at[0, 0]], b0, gs0)

    def body(jj, _):
        c1 = 2 * jj + 1
        pltpu.async_copy(tab.at[ebig.at[c1, 0]], b1, gs1)
        pltpu.make_async_copy(tab.at[ebig.at[0, 0]], b0, gs0).wait()
        s0 = pltpu.async_copy(b0, acc.at[ebig.at[2 * jj, 1]], ss0, add=True)
        pltpu.make_async_copy(tab.at[ebig.at[0, 0]], b1, gs1).wait()
        s1 = pltpu.async_copy(b1, acc.at[ebig.at[c1, 1]], ss1, add=True)
        s0.wait()
        cn = jnp.minimum(2 * jj + 2, nstep - 1)
        pltpu.async_copy(tab.at[ebig.at[cn, 0]], b0, gs0)
        s1.wait()
        return 0

    lax.fori_loop(0, nstep // 2, body, 0)
    pltpu.make_async_copy(tab.at[ebig.at[0, 0]], b0, gs0).wait()


def _make_sc_spmm(cols, preload_idx=False):
    if preload_idx:
        nphase = 1 if cols <= H else 2
        pstep = NSTEP // nphase

        def body(tab, edges3, out, ebig, b0, b1, acc, gs0, gs1, ss0, ss1):
            cid = lax.axis_index("c")
            sid = lax.axis_index("s")
            base = (cid * NS + sid) * NSTEP
            _zero_rows(b0, cols)
            _fill_acc(b0, acc, sid)
            plsc.subcore_barrier()
            for ph in range(nphase):
                pltpu.sync_copy(
                    edges3.at[pl.ds(base + ph * pstep, pstep)], ebig)
                _sc_edge_loop_pre(tab, ebig, acc, b0, b1,
                                  gs0, gs1, ss0, ss1, pstep)
            plsc.subcore_barrier()
            _writeout(acc, b0, out, cid, sid)

        idx_scratch = [pltpu.VMEM((NSTEP // nphase, 2, K), _i32)]
    else:
        def body(tab, edges3, out, e0, e1, b0, b1, acc, gs0, gs1, ss0, ss1):
            cid = lax.axis_index("c")
            sid = lax.axis_index("s")
            _zero_rows(b0, cols)
            _fill_acc(b0, acc, sid)
            plsc.subcore_barrier()
            base = (cid * NS + sid) * NSTEP
            _sc_edge_loop(tab, edges3, acc, e0, e1, b0, b1,
                          gs0, gs1, ss0, ss1, base)
            plsc.subcore_barrier()
            _writeout(acc, b0, out, cid, sid)

        idx_scratch = [pltpu.VMEM((2, K), _i32), pltpu.VMEM((2, K), _i32)]

    return pl.kernel(
        body,
        out_type=jax.ShapeDtypeStruct((NC, NP, cols), _f32),
        mesh=_mesh,
        compiler_params=_CP,
        scratch_types=idx_scratch + [
            pltpu.VMEM((K, cols), _f32),
            pltpu.VMEM((K, cols), _f32),
            pltpu.VMEM_SHARED((NP, cols), _f32),
            pltpu.SemaphoreType.DMA,
            pltpu.SemaphoreType.DMA,
            pltpu.SemaphoreType.DMA,
            pltpu.SemaphoreType.DMA,
        ],
    )


def _sc_gat_body(asm, ad, mv, edges3, xaug, out,
                 e0, e1, av0, av1, dv0, dv1, b0, b1, mvv, acc,
                 gs0, gs1, as0, as1, ds0, ds1, ss0, ss1):
    cid = lax.axis_index("c")
    sid = lax.axis_index("s")
    _zero_rows(b0, GC)
    _fill_acc(b0, acc, sid)
    pltpu.sync_copy(mv, mvv)
    plsc.subcore_barrier()
    base = (cid * NS + sid) * NSTEP
    m = mvv[...]

    def issue(c, e, b, av, dv, gs, asem, dsem):
        pltpu.sync_copy(edges3.at[c], e)
        pltpu.async_copy(xaug.at[e.at[0]], b, gs)
        pltpu.async_copy(asm.at[e.at[0]], av, asem)
        pltpu.async_copy(ad.at[e.at[1]], dv, dsem)

    def tcomp(e, b, av, dv, gs, asem, dsem):
        pltpu.make_async_copy(asm.at[e.at[0]], av, asem).wait()
        pltpu.make_async_copy(ad.at[e.at[1]], dv, dsem).wait()
        for i in range(K // L):
            v = av[pl.ds(i * L, L)] + dv[pl.ds(i * L, L)]
            av[pl.ds(i * L, L)] = jnp.exp(jnp.maximum(v, 0.2 * v) - m)
        pltpu.make_async_copy(xaug.at[e.at[0]], b, gs).wait()

        @plsc.parallel_loop(0, K, unroll=4)
        def _(r):
            t = plsc.load_gather(av, [jnp.full((L,), r, _i32)])
            for c in range(GC // L):
                b[r, pl.ds(c * L, L)] = b[r, pl.ds(c * L, L)] * t

    issue(base, e0, b0, av0, dv0, gs0, as0, ds0)

    def body(jj, _):
        issue(base + 2 * jj + 1, e1, b1, av1, dv1, gs1, as1, ds1)
        tcomp(e0, b0, av0, dv0, gs0, as0, ds0)
        s0 = pltpu.async_copy(b0, acc.at[e0.at[1]], ss0, add=True)
        tcomp(e1, b1, av1, dv1, gs1, as1, ds1)
        s1 = pltpu.async_copy(b1, acc.at[e1.at[1]], ss1, add=True)
        s0.wait()
        cn = base + jnp.minimum(2 * jj + 2, NSTEP - 1)
        issue(cn, e0, b0, av0, dv0, gs0, as0, ds0)
        s1.wait()
        return 0

    lax.fori_loop(0, NSTEP // 2, body, 0)
    pltpu.make_async_copy(xaug.at[e0.at[0]], b0, gs0).wait()
    pltpu.make_async_copy(asm.at[e0.at[0]], av0, as0).wait()
    pltpu.make_async_copy(ad.at[e0.at[1]], dv0, ds0).wait()
    plsc.subcore_barrier()
    _writeout(acc, b0, out, cid, sid)


_sc_gat = pl.kernel(
    _sc_gat_body,
    out_type=jax.ShapeDtypeStruct((NC, NP, GC), _f32),
    mesh=_mesh,
    compiler_params=_CP,
    scratch_types=[
        pltpu.VMEM((2, K), _i32),
        pltpu.VMEM((2, K), _i32),
        pltpu.VMEM((K,), _f32),
        pltpu.VMEM((K,), _f32),
        pltpu.VMEM((K,), _f32),
        pltpu.VMEM((K,), _f32),
        pltpu.VMEM((K, GC), _f32),
        pltpu.VMEM((K, GC), _f32),
        pltpu.VMEM((L,), _f32),
        pltpu.VMEM_SHARED((NP, GC), _f32),
        pltpu.SemaphoreType.DMA,
        pltpu.SemaphoreType.DMA,
        pltpu.SemaphoreType.DMA,
        pltpu.SemaphoreType.DMA,
        pltpu.SemaphoreType.DMA,
        pltpu.SemaphoreType.DMA,
        pltpu.SemaphoreType.DMA,
        pltpu.SemaphoreType.DMA,
    ],
)


_BLK = 2048
_GRID = NP // _BLK


def _rows_spec(cols):
    return pl.BlockSpec((_BLK, cols), lambda i: (i, 0))


def _full_spec(r, c):
    return pl.BlockSpec((r, c), lambda i: (0, 0))


def _tc1a_body(x_ref, kf_ref, w1_ref, xl1_ref):
    xl1_ref[...] = jnp.dot(kf_ref[...] * x_ref[...], w1_ref[...],
                           preferred_element_type=_f32)


_tc1a = pl.pallas_call(
    _tc1a_body,
    grid=(_GRID,),
    in_specs=[_rows_spec(D), _rows_spec(1), _full_spec(D, 2 * H)],
    out_specs=_rows_spec(2 * H),
    out_shape=jax.ShapeDtypeStruct((NP, 2 * H), _f32),
)


def _tc1b_body(xl1_ref, kf_ref, d0_ref, d1_ref, u1_ref, dinv_ref):
    kf = kf_ref[...]
    deg = d0_ref[...][:, 0:1] + d1_ref[...][:, 0:1] + 1.0
    dinv = lax.rsqrt(deg)
    u1_ref[...] = (kf * dinv) * xl1_ref[...]
    dinv_ref[...] = dinv


_tc1b = pl.pallas_call(
    _tc1b_body,
    grid=(_GRID,),
    in_specs=[_rows_spec(2 * H), _rows_spec(1), _rows_spec(16),
              _rows_spec(16)],
    out_specs=[_rows_spec(2 * H), _rows_spec(1)],
    out_shape=[jax.ShapeDtypeStruct((NP, 2 * H), _f32),
               jax.ShapeDtypeStruct((NP, 1), _f32)],
)


def _tc2_body(a0_ref, a1_ref, xl1_ref, dinv_ref, kf_ref, b1_ref, w2_ref,
              xl2_ref, u2_ref):
    dinv = dinv_ref[...]
    kf = kf_ref[...]
    h1 = jax.nn.relu(dinv * (a0_ref[...] + a1_ref[...])
                     + (dinv * dinv) * xl1_ref[...] + b1_ref[...])
    xl2 = jnp.dot(h1, w2_ref[...], preferred_element_type=_f32)
    xl2_ref[...] = xl2
    u2_ref[...] = (kf * dinv) * xl2


_tc2 = pl.pallas_call(
    _tc2_body,
    grid=(_GRID,),
    in_specs=[_rows_spec(2 * H), _rows_spec(2 * H), _rows_spec(2 * H),
              _rows_spec(1), _rows_spec(1), _full_spec(1, 2 * H),
              _full_spec(2 * H, H)],
    out_specs=[_rows_spec(H), _rows_spec(H)],
    out_shape=[jax.ShapeDtypeStruct((NP, H), _f32),
               jax.ShapeDtypeStruct((NP, H), _f32)],
)


def _tc3_body(a0_ref, a1_ref, xl2_ref, dinv_ref, kf_ref, b2_ref, wg_ref,
              asrc_ref, adst_ref,
              xaug_ref, asm_ref, ad_ref, ma_ref, md_ref):
    i = pl.program_id(0)
    dinv = dinv_ref[...]
    kf = kf_ref[...]
    h2 = jax.nn.relu(dinv * (a0_ref[...] + a1_ref[...])
                     + (dinv * dinv) * xl2_ref[...] + b2_ref[...])
    xl3 = jnp.dot(h2, wg_ref[...], preferred_element_type=_f32)
    as_ = jnp.dot(xl3, asrc_ref[...], preferred_element_type=_f32)
    ad_ = jnp.dot(xl3, adst_ref[...], preferred_element_type=_f32)
    asm = jnp.where(kf > 0, as_, -1e30)
    xaug_ref[...] = jnp.concatenate(
        [xl3, jnp.ones((_BLK, 1), _f32), jnp.zeros((_BLK, 15), _f32)], axis=1)
    asm_ref[...] = asm
    ad_ref[...] = ad_

    @pl.when(i == 0)
    def _():
        ma_ref[...] = jnp.full((1, 1), -1e30, _f32)
        md_ref[...] = jnp.full((1, 1), -1e30, _f32)

    ma_ref[...] = jnp.maximum(ma_ref[...], jnp.max(asm))
    md_ref[...] = jnp.maximum(md_ref[...], jnp.max(ad_))


_tc3 = pl.pallas_call(
    _tc3_body,
    grid=(_GRID,),
    in_specs=[_rows_spec(H), _rows_spec(H), _rows_spec(H),
              _rows_spec(1), _rows_spec(1), _full_spec(1, H),
              _full_spec(H, D), _full_spec(D, 1), _full_spec(D, 1)],
    out_specs=[_rows_spec(GC), _rows_spec(1), _rows_spec(1),
               _full_spec(1, 1), _full_spec(1, 1)],
    out_shape=[jax.ShapeDtypeStruct((NP, GC), _f32),
               jax.ShapeDtypeStruct((NP, 1), _f32),
               jax.ShapeDtypeStruct((NP, 1), _f32),
               jax.ShapeDtypeStruct((1, 1), _f32),
               jax.ShapeDtypeStruct((1, 1), _f32)],
)


def _tc4_body(g0_ref, g1_ref, bg_ref, wc_ref, bc_ref, out_ref):
    g = g0_ref[...] + g1_ref[...]
    s = g[:, D:D + 1]
    z = jax.nn.relu(g[:, :D] / (s + 1e-16) + bg_ref[...])
    xr = 1.0 / (1.0 + jnp.exp(-z))
    lg = jnp.dot(xr, wc_ref[...], preferred_element_type=_f32) + bc_ref[...]
    m = jnp.max(lg, axis=1, keepdims=True)
    e = lg - m
    out_ref[...] = e - jnp.log(jnp.sum(jnp.exp(e), axis=1, keepdims=True))


_tc4 = pl.pallas_call(
    _tc4_body,
    grid=(_GRID,),
    in_specs=[_rows_spec(GC), _rows_spec(GC), _full_spec(1, D),
              _full_spec(D, C), _full_spec(1, C)],
    out_specs=_rows_spec(C),
    out_shape=jax.ShapeDtypeStruct((NP, C), _f32),
)

_sc_spmm_deg = _make_sc_spmm(16, preload_idx=True)
_sc_spmm_128 = _make_sc_spmm(2 * H)
_sc_spmm_64 = _make_sc_spmm(H, preload_idx=True)


def kernel(x, edge_index, W1, b1, W2, b2, Wg, att_src, att_dst, bg, Wc, bc):
    with jax.ensure_compile_time_eval():
        perm = jax.random.permutation(jax.random.key(42), N)
        mask_nodes = perm[: int(0.15 * N)]
        keptf = jnp.ones((N,), _f32).at[mask_nodes].set(0.0)
        kf_p = jnp.zeros((NP, 1), _f32).at[:N, 0].set(keptf)
        ktab = jnp.concatenate([kf_p, jnp.zeros((NP, 15), _f32)], axis=1)
    x_p = jnp.zeros((NP, D), _f32).at[:N].set(x)
    ar = jnp.arange(EP - E, dtype=_i32)
    pad = jnp.stack([N + (ar % (NP - N)), ar % N])
    ei = jnp.concatenate([edge_index, pad], axis=1)
    edges3 = jnp.stack(
        [ei[0].reshape(NW * NSTEP, K), ei[1].reshape(NW * NSTEP, K)], axis=1)

    degp = _sc_spmm_deg(ktab, edges3)
    xl1 = _tc1a(x_p, kf_p, W1)
    u1, dinv = _tc1b(xl1, kf_p, degp[0], degp[1])
    agg1 = _sc_spmm_128(u1, edges3)
    xl2, u2 = _tc2(agg1[0], agg1[1], xl1, dinv, kf_p,
                   b1.reshape(1, 2 * H), W2)
    agg2 = _sc_spmm_64(u2, edges3)
    xaug, asm, ad, ma, md = _tc3(agg2[0], agg2[1], xl2, dinv, kf_p,
                                 b2.reshape(1, H), Wg,
                                 att_src.reshape(D, 1), att_dst.reshape(D, 1))
    mglob = jnp.maximum(ma[0, 0] + md[0, 0], 0.0)
    mvec = jnp.full((L,), mglob, _f32)
    gat = _sc_gat(asm.reshape(NP), ad.reshape(NP), mvec, edges3, xaug)
    out = _tc4(gat[0], gat[1], bg.reshape(1, D), Wc, bc.reshape(1, C))
    return out[:N]
